# Initial kernel scaffold; baseline (speedup 1.0000x reference)
import jax
import jax.numpy as jnp
from jax.experimental import pallas as pl


def kernel(x, token_table, pos_table):
    raise NotImplementedError("write your pallas kernel here")


# speedup vs baseline: 2.2160x; 2.2160x over previous
import jax
import jax.numpy as jnp
from jax import lax
from jax.experimental import pallas as pl
from jax.experimental.pallas import tpu as pltpu
from jax.experimental.pallas import tpu_sc as plsc

VOCAB = 100000
MAX_LEN = 200
EMBED_DIM = 128
BATCH = 4096

NUM_CORES = 2
NUM_SUBCORES = 16
NUM_WORKERS = NUM_CORES * NUM_SUBCORES
TOTAL_ROWS = BATCH * MAX_LEN
ROWS_PER_WORKER = TOTAL_ROWS // NUM_WORKERS
IDX_MINOR = 128
CHUNK = 2 * IDX_MINOR
SUPER = 8 * IDX_MINOR
SUPERS_PER_WORKER = ROWS_PER_WORKER // SUPER
CHUNKS_PER_SUPER = SUPER // CHUNK
LANES = 16


def _sc_body(x_hbm, tok_hbm, pos_hbm, out_hbm, idx_v, rows_v, pos_v, sem):
    wid = lax.axis_index("s") * NUM_CORES + lax.axis_index("c")
    base_row = wid * ROWS_PER_WORKER

    pltpu.sync_copy(pos_hbm, pos_v)

    def super_body(s, _):
        srow = base_row + s * SUPER
        pltpu.sync_copy(x_hbm.at[srow // SUPER], idx_v)

        for cc in range(CHUNKS_PER_SUPER):
            row0 = srow + cc * CHUNK
            cp0 = pltpu.make_async_copy(
                tok_hbm.at[idx_v.at[2 * cc]],
                rows_v.at[pl.ds(0, IDX_MINOR)], sem)
            cp1 = pltpu.make_async_copy(
                tok_hbm.at[idx_v.at[2 * cc + 1]],
                rows_v.at[pl.ds(IDX_MINOR, IDX_MINOR)], sem)
            cp0.start()
            cp1.start()
            cp0.wait()
            cp1.wait()

            l0 = row0 % MAX_LEN

            def add_row(i, _):
                l = lax.rem(l0 + i, MAX_LEN)
                for j in range(EMBED_DIM // LANES):
                    sl = pl.ds(j * LANES, LANES)
                    rows_v[i, sl] = rows_v[i, sl] + pos_v[l, sl]
                return 0

            lax.fori_loop(0, CHUNK, add_row, 0)

            pltpu.sync_copy(rows_v, out_hbm.at[pl.ds(row0, CHUNK)])
        return 0

    lax.fori_loop(0, SUPERS_PER_WORKER, super_body, 0)


@jax.jit
def _embed(x3d, token_table, pos_table):
    mesh = plsc.VectorSubcoreMesh(
        core_axis_name="c", subcore_axis_name="s",
        num_cores=NUM_CORES, num_subcores=NUM_SUBCORES)
    fn = pl.kernel(
        _sc_body,
        out_type=jax.ShapeDtypeStruct((TOTAL_ROWS, EMBED_DIM), jnp.float32),
        mesh=mesh,
        scratch_types=[
            pltpu.VMEM((8, IDX_MINOR), jnp.int32),
            pltpu.VMEM((CHUNK, EMBED_DIM), jnp.float32),
            pltpu.VMEM((MAX_LEN, EMBED_DIM), jnp.float32),
            pltpu.SemaphoreType.DMA,
        ],
    )
    return fn(x3d, token_table, pos_table)


def kernel(x, token_table, pos_table):
    x3d = x.reshape(TOTAL_ROWS // SUPER, 8, IDX_MINOR).astype(jnp.int32)
    out = _embed(x3d, token_table, pos_table)
    return out.reshape(BATCH, MAX_LEN, EMBED_DIM)


# speedup vs baseline: 2.9419x; 1.3276x over previous
import jax
import jax.numpy as jnp
from jax import lax
from jax.experimental import pallas as pl
from jax.experimental.pallas import tpu as pltpu
from jax.experimental.pallas import tpu_sc as plsc

VOCAB = 100000
MAX_LEN = 200
EMBED_DIM = 128
BATCH = 4096

NUM_CORES = 2
NUM_SUBCORES = 16
NUM_WORKERS = NUM_CORES * NUM_SUBCORES
TOTAL_ROWS = BATCH * MAX_LEN
ROWS_PER_WORKER = TOTAL_ROWS // NUM_WORKERS
CHUNK = 128
NUM_CHUNKS = ROWS_PER_WORKER // CHUNK
NBUF = 4
LANES = 16


def _sc_body(x_hbm, tok_hbm, pos_hbm, out_hbm, idx_v, rows_v, pos_v,
             sem_g, sem_o):
    wid = lax.axis_index("s") * NUM_CORES + lax.axis_index("c")
    base_row = wid * ROWS_PER_WORKER

    pltpu.sync_copy(pos_hbm, pos_v)
    pltpu.sync_copy(x_hbm.at[wid], idx_v)

    def gather(c, b):
        return pltpu.make_async_copy(
            tok_hbm.at[idx_v.at[c]], rows_v.at[b], sem_g)

    def outcp(c, b):
        return pltpu.make_async_copy(
            rows_v.at[b], out_hbm.at[pl.ds(base_row + c * CHUNK, CHUNK)],
            sem_o)

    gather(0, 0).start()

    def ring_body(t, _):
        for k in range(NBUF):
            c = NBUF * t + k
            nb = (k + 1) % NBUF

            @pl.when(c >= NBUF - 1)
            def _():
                outcp(c - (NBUF - 1), nb).wait()

            @pl.when(c + 1 < NUM_CHUNKS)
            def _():
                gather(c + 1, nb).start()

            gather(c, k).wait()

            l0 = lax.rem(c * CHUNK, MAX_LEN)

            def add_row(i, _):
                l = lax.rem(l0 + i, MAX_LEN)
                for j in range(EMBED_DIM // LANES):
                    sl = pl.ds(j * LANES, LANES)
                    rows_v[k, i, sl] = rows_v[k, i, sl] + pos_v[l, sl]
                return 0

            lax.fori_loop(0, CHUNK, add_row, 0)

            outcp(c, k).start()
        return 0

    lax.fori_loop(0, NUM_CHUNKS // NBUF, ring_body, 0)

    for c in range(NUM_CHUNKS - (NBUF - 1), NUM_CHUNKS):
        outcp(c, c % NBUF).wait()


@jax.jit
def _embed(x3d, token_table, pos_table):
    mesh = plsc.VectorSubcoreMesh(
        core_axis_name="c", subcore_axis_name="s",
        num_cores=NUM_CORES, num_subcores=NUM_SUBCORES)
    fn = pl.kernel(
        _sc_body,
        out_type=jax.ShapeDtypeStruct((TOTAL_ROWS, EMBED_DIM), jnp.float32),
        mesh=mesh,
        scratch_types=[
            pltpu.VMEM((NUM_CHUNKS, CHUNK), jnp.int32),
            pltpu.VMEM((NBUF, CHUNK, EMBED_DIM), jnp.float32),
            pltpu.VMEM((MAX_LEN, EMBED_DIM), jnp.float32),
            pltpu.SemaphoreType.DMA,
            pltpu.SemaphoreType.DMA,
        ],
    )
    return fn(x3d, token_table, pos_table)


def kernel(x, token_table, pos_table):
    x3d = x.reshape(NUM_WORKERS, NUM_CHUNKS, CHUNK).astype(jnp.int32)
    out = _embed(x3d, token_table, pos_table)
    return out.reshape(BATCH, MAX_LEN, EMBED_DIM)


# speedup vs baseline: 9.0628x; 3.0806x over previous
import jax
import jax.numpy as jnp
from jax import lax
from jax.experimental import pallas as pl
from jax.experimental.pallas import tpu as pltpu
from jax.experimental.pallas import tpu_sc as plsc

VOCAB = 100000
MAX_LEN = 200
EMBED_DIM = 128
BATCH = 4096

NUM_CORES = 2
NUM_SUBCORES = 16
NUM_WORKERS = NUM_CORES * NUM_SUBCORES
TOTAL_ROWS = BATCH * MAX_LEN
ROWS_PER_WORKER = TOTAL_ROWS // NUM_WORKERS
CHUNK = MAX_LEN
NUM_CHUNKS = ROWS_PER_WORKER // CHUNK
G0 = 128
G1 = CHUNK - G0
NBUF = 3
LANES = 16


def _sc_body(x_hbm, tok_hbm, pos_hbm, out_hbm, idx_v, rows_v, pos_v,
             sem_g, sem_o):
    wid = lax.axis_index("s") * NUM_CORES + lax.axis_index("c")
    base_row = pl.multiple_of(wid * ROWS_PER_WORKER, ROWS_PER_WORKER)

    pltpu.sync_copy(x_hbm.at[pl.ds(base_row, ROWS_PER_WORKER)], idx_v)
    pltpu.sync_copy(pos_hbm, pos_v)

    def gather(c, b):
        off = pl.multiple_of(c * CHUNK, 8)
        return (
            pltpu.make_async_copy(
                tok_hbm.at[idx_v.at[pl.ds(off, G0)]],
                rows_v.at[b].at[pl.ds(0, G0)], sem_g),
            pltpu.make_async_copy(
                tok_hbm.at[idx_v.at[pl.ds(off + G0, G1)]],
                rows_v.at[b].at[pl.ds(G0, G1)], sem_g),
        )

    def gather_start(c, b):
        cp0, cp1 = gather(c, b)
        cp0.start()
        cp1.start()

    def gather_wait(c, b):
        cp0, cp1 = gather(c, b)
        cp0.wait()
        cp1.wait()

    def outcp(c, b):
        return pltpu.make_async_copy(
            rows_v.at[b], out_hbm.at[pl.ds(base_row + c * CHUNK, CHUNK)],
            sem_o)

    def add_chunk(b):
        def add_row(i, _):
            for j in range(EMBED_DIM // LANES):
                sl = pl.ds(j * LANES, LANES)
                plsc.addupdate(rows_v.at[b, i, sl], pos_v[i, sl])
            return 0

        lax.fori_loop(0, CHUNK, add_row, 0)

    gather_start(0, 0)

    def ring_body(t, _):
        for k in range(NBUF):
            c = NBUF * t + k
            nb = (k + 1) % NBUF

            @pl.when(c >= NBUF - 1)
            def _():
                outcp(c - (NBUF - 1), nb).wait()

            gather_start(c + 1, nb)
            gather_wait(c, k)
            add_chunk(k)
            outcp(c, k).start()
        return 0

    body_chunks = NUM_CHUNKS - 2
    lax.fori_loop(0, body_chunks // NBUF, ring_body, 0)

    outcp(NUM_CHUNKS - 4, 1).wait()
    gather_start(NUM_CHUNKS - 1, 1)
    gather_wait(NUM_CHUNKS - 2, 0)
    add_chunk(0)
    outcp(NUM_CHUNKS - 2, 0).start()

    outcp(NUM_CHUNKS - 3, 2).wait()
    gather_wait(NUM_CHUNKS - 1, 1)
    add_chunk(1)
    outcp(NUM_CHUNKS - 1, 1).start()

    outcp(NUM_CHUNKS - 2, 0).wait()
    outcp(NUM_CHUNKS - 1, 1).wait()


@jax.jit
def _embed(x1d, token_table, pos_table):
    mesh = plsc.VectorSubcoreMesh(
        core_axis_name="c", subcore_axis_name="s",
        num_cores=NUM_CORES, num_subcores=NUM_SUBCORES)
    fn = pl.kernel(
        _sc_body,
        out_type=jax.ShapeDtypeStruct((TOTAL_ROWS, EMBED_DIM), jnp.float32),
        mesh=mesh,
        scratch_types=[
            pltpu.VMEM((ROWS_PER_WORKER,), jnp.int32),
            pltpu.VMEM((NBUF, CHUNK, EMBED_DIM), jnp.float32),
            pltpu.VMEM((MAX_LEN, EMBED_DIM), jnp.float32),
            pltpu.SemaphoreType.DMA,
            pltpu.SemaphoreType.DMA,
        ],
    )
    return fn(x1d, token_table, pos_table)


def kernel(x, token_table, pos_table):
    x1d = x.reshape(TOTAL_ROWS).astype(jnp.int32)
    out = _embed(x1d, token_table, pos_table)
    return out.reshape(BATCH, MAX_LEN, EMBED_DIM)
